# Initial kernel scaffold; baseline (speedup 1.0000x reference)
import jax
import jax.numpy as jnp
from jax.experimental import pallas as pl


def kernel(x, edge_index, W):
    raise NotImplementedError("write your pallas kernel here")


# speedup vs baseline: 4.0348x; 4.0348x over previous
import functools

import jax
import jax.numpy as jnp
from jax import lax
from jax.experimental import pallas as pl
from jax.experimental.pallas import tpu as pltpu
from jax.experimental.pallas import tpu_sc as plsc

N_NODES = 10000
N_EDGES = 320000
D_FEAT = 128
EMBED_DIM = 128

NC = 2
NS = 16
NW = NC * NS
K = 64
NCH = 160
EPAD = NW * NCH * K
NPAD = N_NODES
ROWS_PT = NPAD // NS
CW = 8
NPE = EPAD - N_EDGES
CZR = 2000


def _sc_body(x_hbm, srcr, dstr, rid, cz, agg_out, cnt_out,
             ids_v, src_v, dst_v, rows0, rows1, ones_v, sem0, sem1,
             acc_sh, cnt_sh):
    c = lax.axis_index("c")
    s = lax.axis_index("s")
    w = c * NS + s

    def zvec(i, carry):
        rows0[i // 8, pl.ds((i % 8) * 16, 16)] = jnp.zeros((16,), jnp.float32)
        return carry
    lax.fori_loop(0, K * 8, zvec, 0)

    def zcp(j, carry):
        pltpu.sync_copy(rows0, acc_sh.at[pl.ds(s * ROWS_PT + j * K, K)])
        return carry
    lax.fori_loop(0, ROWS_PT // K, zcp, 0)
    pltpu.sync_copy(rows0.at[pl.ds(0, ROWS_PT - (ROWS_PT // K) * K)],
                    acc_sh.at[pl.ds(s * ROWS_PT + (ROWS_PT // K) * K,
                                    ROWS_PT - (ROWS_PT // K) * K)])

    pltpu.sync_copy(cz.at[pl.ds(0, K)], ones_v)
    @pl.when(s < NPAD // CZR)
    def _():
        pltpu.sync_copy(cz.at[pl.ds(K, CZR)],
                        cnt_sh.at[pl.ds(s * CZR, CZR)])

    pltpu.sync_copy(rid.at[w], ids_v)
    pltpu.async_copy(srcr.at[ids_v.at[0]], src_v.at[pl.ds(0, NCH // 2)],
                     sem0).wait()
    pltpu.async_copy(srcr.at[ids_v.at[1]], src_v.at[pl.ds(NCH // 2, NCH // 2)],
                     sem0).wait()
    pltpu.async_copy(dstr.at[ids_v.at[0]], dst_v.at[pl.ds(0, NCH // 2)],
                     sem0).wait()
    pltpu.async_copy(dstr.at[ids_v.at[1]], dst_v.at[pl.ds(NCH // 2, NCH // 2)],
                     sem0).wait()
    plsc.subcore_barrier()

    pltpu.make_async_copy(x_hbm.at[src_v.at[0]], rows0, sem0).start()

    def pair(i, carry):
        j = 2 * i
        pltpu.make_async_copy(x_hbm.at[src_v.at[j + 1]], rows1, sem1).start()
        pltpu.make_async_copy(x_hbm.at[src_v.at[j]], rows0, sem0).wait()
        pltpu.sync_copy(rows0, acc_sh.at[dst_v.at[j]], add=True)
        pltpu.sync_copy(ones_v, cnt_sh.at[dst_v.at[j]], add=True)
        pltpu.make_async_copy(x_hbm.at[src_v.at[j + 2]], rows0, sem0).start()
        pltpu.make_async_copy(x_hbm.at[src_v.at[j + 1]], rows1, sem1).wait()
        pltpu.sync_copy(rows1, acc_sh.at[dst_v.at[j + 1]], add=True)
        pltpu.sync_copy(ones_v, cnt_sh.at[dst_v.at[j + 1]], add=True)
        return carry

    lax.fori_loop(0, NCH // 2 - 1, pair, 0)
    pltpu.make_async_copy(x_hbm.at[src_v.at[NCH - 1]], rows1, sem1).start()
    pltpu.make_async_copy(x_hbm.at[src_v.at[NCH - 2]], rows0, sem0).wait()
    pltpu.sync_copy(rows0, acc_sh.at[dst_v.at[NCH - 2]], add=True)
    pltpu.sync_copy(ones_v, cnt_sh.at[dst_v.at[NCH - 2]], add=True)
    pltpu.make_async_copy(x_hbm.at[src_v.at[NCH - 1]], rows1, sem1).wait()
    pltpu.sync_copy(rows1, acc_sh.at[dst_v.at[NCH - 1]], add=True)
    pltpu.sync_copy(ones_v, cnt_sh.at[dst_v.at[NCH - 1]], add=True)

    plsc.subcore_barrier()
    pltpu.sync_copy(acc_sh.at[pl.ds(s * ROWS_PT, ROWS_PT)],
                    agg_out.at[c, pl.ds(s * ROWS_PT, ROWS_PT)])
    @pl.when(s < NPAD // CZR)
    def _():
        pltpu.sync_copy(cnt_sh.at[pl.ds(s * CZR, CZR)],
                        cnt_out.at[c, pl.ds(s * CZR, CZR)])


@functools.cache
def _sc_agg():
    return pl.kernel(
        _sc_body,
        out_type=(jax.ShapeDtypeStruct((NC, NPAD, D_FEAT), jnp.float32),
                  jax.ShapeDtypeStruct((NC, NPAD, CW), jnp.float32)),
        mesh=plsc.VectorSubcoreMesh(core_axis_name="c", subcore_axis_name="s",
                                    num_cores=NC, num_subcores=NS),
        scratch_types=[
            pltpu.VMEM((2, NCH // 2), jnp.int32),
            pltpu.VMEM((NCH, K), jnp.int32),
            pltpu.VMEM((NCH, K), jnp.int32),
            pltpu.VMEM((K, D_FEAT), jnp.float32),
            pltpu.VMEM((K, D_FEAT), jnp.float32),
            pltpu.VMEM((K, CW), jnp.float32),
            pltpu.SemaphoreType.DMA,
            pltpu.SemaphoreType.DMA,
            pltpu.VMEM_SHARED((NPAD, D_FEAT), jnp.float32),
            pltpu.VMEM_SHARED((NPAD, CW), jnp.float32),
        ],
        compiler_params=pltpu.CompilerParams(use_tc_tiling_on_sc=False),
    )


_TC_B = 1024


def _tc_body(x_ref, agg_ref, cnt_ref, w_ref, o_ref):
    a = agg_ref[0] + agg_ref[1]
    cnt = cnt_ref[0, :, 0:1] + cnt_ref[1, :, 0:1]
    rid = lax.broadcasted_iota(jnp.int32, (_TC_B, 1), 0)
    m = jnp.where((rid == 0) & (pl.program_id(0) == 0), float(NPE), 0.0)
    a = a - m * x_ref[...]
    cnt = cnt - m
    neigh = a / jnp.maximum(cnt, 1.0)
    w1 = w_ref[:, :D_FEAT]
    w2 = w_ref[:, D_FEAT:]
    dn = (((1,), (1,)), ((), ()))
    o = lax.dot_general(w1, x_ref[...], dn,
                        preferred_element_type=jnp.float32,
                        precision=lax.Precision.HIGHEST)
    o = o + lax.dot_general(w2, neigh, dn,
                            preferred_element_type=jnp.float32,
                            precision=lax.Precision.HIGHEST)
    o_ref[...] = jnp.maximum(o, 0.0)


def _tc_combine(x, agg, cntv, W):
    grid = (pl.cdiv(N_NODES, _TC_B),)
    return pl.pallas_call(
        _tc_body,
        grid=grid,
        in_specs=[
            pl.BlockSpec((_TC_B, D_FEAT), lambda i: (i, 0)),
            pl.BlockSpec((NC, _TC_B, D_FEAT), lambda i: (0, i, 0)),
            pl.BlockSpec((NC, _TC_B, CW), lambda i: (0, i, 0)),
            pl.BlockSpec((EMBED_DIM, 2 * D_FEAT), lambda i: (0, 0)),
        ],
        out_specs=pl.BlockSpec((EMBED_DIM, _TC_B), lambda i: (0, i)),
        out_shape=jax.ShapeDtypeStruct((EMBED_DIM, N_NODES), jnp.float32),
    )(x, agg, cntv, W)


def kernel(x, edge_index, W):
    zpad = jnp.zeros((NPE,), jnp.int32)
    src = jnp.concatenate([edge_index[0], zpad]).reshape(EPAD // K, K)
    dst = jnp.concatenate([edge_index[1], zpad]).reshape(EPAD // K, K)
    rid = jnp.arange(NW * NCH, dtype=jnp.int32).reshape(NW, 2, NCH // 2)
    cz = jnp.concatenate(
        [jnp.ones((K, CW), jnp.float32),
         jnp.zeros((CZR, CW), jnp.float32)], axis=0)
    agg, cntv = _sc_agg()(x, src, dst, rid, cz)
    return _tc_combine(x, agg, cntv, W)


# speedup vs baseline: 4.0357x; 1.0002x over previous
import functools

import jax
import jax.numpy as jnp
from jax import lax
from jax.experimental import pallas as pl
from jax.experimental.pallas import tpu as pltpu
from jax.experimental.pallas import tpu_sc as plsc

N_NODES = 10000
N_EDGES = 320000
D_FEAT = 128
EMBED_DIM = 128

NC = 2
NS = 16
NW = NC * NS
K = 64
NCH = 160
EPAD = NW * NCH * K
NPAD = N_NODES
ROWS_PT = NPAD // NS
CW = 8
NPE = EPAD - N_EDGES
CZR = 2000


def _sc_body(x_hbm, srcr, dstr, rid, cz, agg_out, cnt_out,
             ids_v, src_v, dst_v, rows0, rows1, ones_v, sem0, sem1,
             acc_sh, cnt_sh):
    c = lax.axis_index("c")
    s = lax.axis_index("s")
    w = c * NS + s

    def zvec(i, carry):
        rows0[i // 8, pl.ds((i % 8) * 16, 16)] = jnp.zeros((16,), jnp.float32)
        return carry
    lax.fori_loop(0, K * 8, zvec, 0)

    def zcp(j, carry):
        pltpu.sync_copy(rows0, acc_sh.at[pl.ds(s * ROWS_PT + j * K, K)])
        return carry
    lax.fori_loop(0, ROWS_PT // K, zcp, 0)
    pltpu.sync_copy(rows0.at[pl.ds(0, ROWS_PT - (ROWS_PT // K) * K)],
                    acc_sh.at[pl.ds(s * ROWS_PT + (ROWS_PT // K) * K,
                                    ROWS_PT - (ROWS_PT // K) * K)])

    pltpu.sync_copy(cz.at[pl.ds(0, K)], ones_v)
    @pl.when(s < NPAD // CZR)
    def _():
        pltpu.sync_copy(cz.at[pl.ds(K, CZR)],
                        cnt_sh.at[pl.ds(s * CZR, CZR)])

    pltpu.sync_copy(rid.at[w], ids_v)
    pltpu.async_copy(srcr.at[ids_v.at[0]], src_v.at[pl.ds(0, NCH // 2)],
                     sem0).wait()
    pltpu.async_copy(srcr.at[ids_v.at[1]], src_v.at[pl.ds(NCH // 2, NCH // 2)],
                     sem0).wait()
    pltpu.async_copy(dstr.at[ids_v.at[0]], dst_v.at[pl.ds(0, NCH // 2)],
                     sem0).wait()
    pltpu.async_copy(dstr.at[ids_v.at[1]], dst_v.at[pl.ds(NCH // 2, NCH // 2)],
                     sem0).wait()
    plsc.subcore_barrier()

    pltpu.make_async_copy(x_hbm.at[src_v.at[0]], rows0, sem0).start()

    def pair(i, carry):
        j = 2 * i
        pltpu.make_async_copy(x_hbm.at[src_v.at[j + 1]], rows1, sem1).start()
        pltpu.make_async_copy(x_hbm.at[src_v.at[j]], rows0, sem0).wait()
        pltpu.sync_copy(rows0, acc_sh.at[dst_v.at[j]], add=True)
        pltpu.sync_copy(ones_v, cnt_sh.at[dst_v.at[j]], add=True)
        pltpu.make_async_copy(x_hbm.at[src_v.at[j + 2]], rows0, sem0).start()
        pltpu.make_async_copy(x_hbm.at[src_v.at[j + 1]], rows1, sem1).wait()
        pltpu.sync_copy(rows1, acc_sh.at[dst_v.at[j + 1]], add=True)
        pltpu.sync_copy(ones_v, cnt_sh.at[dst_v.at[j + 1]], add=True)
        return carry

    lax.fori_loop(0, NCH // 2 - 1, pair, 0)
    pltpu.make_async_copy(x_hbm.at[src_v.at[NCH - 1]], rows1, sem1).start()
    pltpu.make_async_copy(x_hbm.at[src_v.at[NCH - 2]], rows0, sem0).wait()
    pltpu.sync_copy(rows0, acc_sh.at[dst_v.at[NCH - 2]], add=True)
    pltpu.sync_copy(ones_v, cnt_sh.at[dst_v.at[NCH - 2]], add=True)
    pltpu.make_async_copy(x_hbm.at[src_v.at[NCH - 1]], rows1, sem1).wait()
    pltpu.sync_copy(rows1, acc_sh.at[dst_v.at[NCH - 1]], add=True)
    pltpu.sync_copy(ones_v, cnt_sh.at[dst_v.at[NCH - 1]], add=True)

    plsc.subcore_barrier()
    pltpu.sync_copy(acc_sh.at[pl.ds(s * ROWS_PT, ROWS_PT)],
                    agg_out.at[c, pl.ds(s * ROWS_PT, ROWS_PT)])
    @pl.when(s < NPAD // CZR)
    def _():
        pltpu.sync_copy(cnt_sh.at[pl.ds(s * CZR, CZR)],
                        cnt_out.at[c, pl.ds(s * CZR, CZR)])


@functools.cache
def _sc_agg():
    return pl.kernel(
        _sc_body,
        out_type=(jax.ShapeDtypeStruct((NC, NPAD, D_FEAT), jnp.float32),
                  jax.ShapeDtypeStruct((NC, NPAD, CW), jnp.float32)),
        mesh=plsc.VectorSubcoreMesh(core_axis_name="c", subcore_axis_name="s",
                                    num_cores=NC, num_subcores=NS),
        scratch_types=[
            pltpu.VMEM((2, NCH // 2), jnp.int32),
            pltpu.VMEM((NCH, K), jnp.int32),
            pltpu.VMEM((NCH, K), jnp.int32),
            pltpu.VMEM((K, D_FEAT), jnp.float32),
            pltpu.VMEM((K, D_FEAT), jnp.float32),
            pltpu.VMEM((K, CW), jnp.float32),
            pltpu.SemaphoreType.DMA,
            pltpu.SemaphoreType.DMA,
            pltpu.VMEM_SHARED((NPAD, D_FEAT), jnp.float32),
            pltpu.VMEM_SHARED((NPAD, CW), jnp.float32),
        ],
        compiler_params=pltpu.CompilerParams(use_tc_tiling_on_sc=False),
    )


_TC_B = 1024


def _tc_body(x_ref, x0_ref, agg_ref, cnt_ref, w_ref, o_ref):
    a = agg_ref[0] + agg_ref[1]
    cnt = cnt_ref[0, :, 0:1] + cnt_ref[1, :, 0:1]
    rid = (lax.broadcasted_iota(jnp.int32, (_TC_B, 1), 0)
           + pl.program_id(0) * _TC_B)
    m = jnp.where(rid < NPE, 1.0, 0.0)
    a = a - m * x0_ref[...]
    cnt = cnt - m
    neigh = a / jnp.maximum(cnt, 1.0)
    w1 = w_ref[:, :D_FEAT]
    w2 = w_ref[:, D_FEAT:]
    dn = (((1,), (1,)), ((), ()))
    o = lax.dot_general(w1, x_ref[...], dn,
                        preferred_element_type=jnp.float32,
                        precision=lax.Precision.HIGHEST)
    o = o + lax.dot_general(w2, neigh, dn,
                            preferred_element_type=jnp.float32,
                            precision=lax.Precision.HIGHEST)
    o_ref[...] = jnp.maximum(o, 0.0)


def _tc_combine(x, agg, cntv, W):
    grid = (pl.cdiv(N_NODES, _TC_B),)
    return pl.pallas_call(
        _tc_body,
        grid=grid,
        in_specs=[
            pl.BlockSpec((_TC_B, D_FEAT), lambda i: (i, 0)),
            pl.BlockSpec((1, D_FEAT), lambda i: (0, 0)),
            pl.BlockSpec((NC, _TC_B, D_FEAT), lambda i: (0, i, 0)),
            pl.BlockSpec((NC, _TC_B, CW), lambda i: (0, i, 0)),
            pl.BlockSpec((EMBED_DIM, 2 * D_FEAT), lambda i: (0, 0)),
        ],
        out_specs=pl.BlockSpec((EMBED_DIM, _TC_B), lambda i: (0, i)),
        out_shape=jax.ShapeDtypeStruct((EMBED_DIM, N_NODES), jnp.float32),
    )(x, x[0:1], agg, cntv, W)


def kernel(x, edge_index, W):
    zpad = jnp.zeros((NPE,), jnp.int32)
    src = jnp.concatenate([edge_index[0], zpad]).reshape(EPAD // K, K)
    dst = jnp.concatenate(
        [edge_index[1], jnp.arange(NPE, dtype=jnp.int32)]).reshape(EPAD // K, K)
    rid = jnp.arange(NW * NCH, dtype=jnp.int32).reshape(NW, 2, NCH // 2)
    cz = jnp.concatenate(
        [jnp.ones((K, CW), jnp.float32),
         jnp.zeros((CZR, CW), jnp.float32)], axis=0)
    agg, cntv = _sc_agg()(x, src, dst, rid, cz)
    return _tc_combine(x, agg, cntv, W)


# speedup vs baseline: 10.1875x; 2.5243x over previous
import functools

import jax
import jax.numpy as jnp
from jax import lax
from jax.experimental import pallas as pl
from jax.experimental.pallas import tpu as pltpu
from jax.experimental.pallas import tpu_sc as plsc

N_NODES = 10000
N_EDGES = 320000
D_FEAT = 128
EMBED_DIM = 128

DA = 136
NC = 2
NS = 16
NW = NC * NS
EPW = N_EDGES // NW
K = 80
NCH = EPW // K
NPAD = N_NODES
ROWS_PT = NPAD // NS


def _sc_body(xa, srcr, dstr, acc_out,
             src_v, dst_v, rows0, rows1, sem0, sem1, acc_sh):
    c = lax.axis_index("c")
    s = lax.axis_index("s")
    w = c * NS + s

    def zvec(i, carry):
        col = jnp.minimum((i % 9) * 16, DA - 16)
        rows0[i // 9, pl.ds(col, 16)] = jnp.zeros((16,), jnp.float32)
        return carry
    lax.fori_loop(0, K * 9, zvec, 0)

    def zcp(j, carry):
        pltpu.sync_copy(rows0, acc_sh.at[pl.ds(s * ROWS_PT + j * K, K)])
        return carry
    lax.fori_loop(0, ROWS_PT // K, zcp, 0)
    pltpu.sync_copy(rows0.at[pl.ds(0, ROWS_PT - (ROWS_PT // K) * K)],
                    acc_sh.at[pl.ds(s * ROWS_PT + (ROWS_PT // K) * K,
                                    ROWS_PT - (ROWS_PT // K) * K)])

    pltpu.sync_copy(srcr.at[w], src_v)
    pltpu.sync_copy(dstr.at[w], dst_v)
    plsc.subcore_barrier()

    pltpu.make_async_copy(xa.at[src_v.at[0]], rows0, sem0).start()

    def pair(i, carry):
        j = 2 * i
        pltpu.make_async_copy(xa.at[src_v.at[j + 1]], rows1, sem1).start()
        pltpu.make_async_copy(xa.at[src_v.at[j]], rows0, sem0).wait()
        pltpu.sync_copy(rows0, acc_sh.at[dst_v.at[j]], add=True)
        pltpu.make_async_copy(xa.at[src_v.at[j + 2]], rows0, sem0).start()
        pltpu.make_async_copy(xa.at[src_v.at[j + 1]], rows1, sem1).wait()
        pltpu.sync_copy(rows1, acc_sh.at[dst_v.at[j + 1]], add=True)
        return carry

    lax.fori_loop(0, (NCH - 1) // 2, pair, 0)
    pltpu.make_async_copy(xa.at[src_v.at[NCH - 1]], rows0, sem0).wait()
    pltpu.sync_copy(rows0, acc_sh.at[dst_v.at[NCH - 1]], add=True)

    plsc.subcore_barrier()
    pltpu.sync_copy(acc_sh.at[pl.ds(s * ROWS_PT, ROWS_PT)],
                    acc_out.at[c, pl.ds(s * ROWS_PT, ROWS_PT)])


@functools.cache
def _sc_agg():
    return pl.kernel(
        _sc_body,
        out_type=jax.ShapeDtypeStruct((NC, NPAD, DA), jnp.float32),
        mesh=plsc.VectorSubcoreMesh(core_axis_name="c", subcore_axis_name="s",
                                    num_cores=NC, num_subcores=NS),
        scratch_types=[
            pltpu.VMEM((NCH, K), jnp.int32),
            pltpu.VMEM((NCH, K), jnp.int32),
            pltpu.VMEM((K, DA), jnp.float32),
            pltpu.VMEM((K, DA), jnp.float32),
            pltpu.SemaphoreType.DMA,
            pltpu.SemaphoreType.DMA,
            pltpu.VMEM_SHARED((NPAD, DA), jnp.float32),
        ],
        compiler_params=pltpu.CompilerParams(use_tc_tiling_on_sc=False),
    )


_TC_B = 1024


def _tc_body(x_ref, acc_ref, w_ref, o_ref):
    a = acc_ref[0] + acc_ref[1]
    cnt = jnp.maximum(a[:, D_FEAT:D_FEAT + 1], 1.0)
    neigh = a[:, :D_FEAT] / cnt
    w1 = w_ref[:, :D_FEAT]
    w2 = w_ref[:, D_FEAT:]
    dn = (((1,), (1,)), ((), ()))
    o = lax.dot_general(w1, x_ref[...], dn,
                        preferred_element_type=jnp.float32,
                        precision=lax.Precision.HIGHEST)
    o = o + lax.dot_general(w2, neigh, dn,
                            preferred_element_type=jnp.float32,
                            precision=lax.Precision.HIGHEST)
    o_ref[...] = jnp.maximum(o, 0.0)


def _tc_combine(x, acc, W):
    grid = (pl.cdiv(N_NODES, _TC_B),)
    return pl.pallas_call(
        _tc_body,
        grid=grid,
        in_specs=[
            pl.BlockSpec((_TC_B, D_FEAT), lambda i: (i, 0)),
            pl.BlockSpec((NC, _TC_B, DA), lambda i: (0, i, 0)),
            pl.BlockSpec((EMBED_DIM, 2 * D_FEAT), lambda i: (0, 0)),
        ],
        out_specs=pl.BlockSpec((EMBED_DIM, _TC_B), lambda i: (0, i)),
        out_shape=jax.ShapeDtypeStruct((EMBED_DIM, N_NODES), jnp.float32),
    )(x, acc, W)


def kernel(x, edge_index, W):
    xa = jnp.concatenate(
        [x, jnp.ones((N_NODES, 1), jnp.float32),
         jnp.zeros((N_NODES, DA - D_FEAT - 1), jnp.float32)], axis=1)
    srcr = edge_index[0].reshape(NW, NCH, K)
    dstr = edge_index[1].reshape(NW, NCH, K)
    acc = _sc_agg()(xa, srcr, dstr)
    return _tc_combine(x, acc, W)
